# baseline (speedup 1.0000x reference)
import functools

import jax
import jax.numpy as jnp
from jax import lax
from jax.experimental import pallas as pl
from jax.experimental.pallas import tpu as pltpu
from jax.experimental.pallas import tpu_sc as plsc

N = 10000
E = 320000
D = 128
H_DIM = 128
C = 64

_L = 16


def _make_spmm(n, e, d):
  info = plsc.get_sparse_core_info()
  nc, ns = info.num_cores, info.num_subcores
  nw = nc * ns
  e_per_w = e // nw
  K = 128
  n_chunks = -(-e_per_w // K)
  NBUF = 2
  ROWCH = 80
  n_row_chunks = n // ROWCH
  max_k = -(-n_row_chunks // ns)

  mesh = plsc.VectorSubcoreMesh(core_axis_name="c", subcore_axis_name="s")

  @functools.partial(
      pl.kernel,
      out_type=jax.ShapeDtypeStruct((nc, n, d), jnp.float32),
      mesh=mesh,
      scratch_types=[
          pltpu.VMEM((n_chunks, K), jnp.int32),
          pltpu.VMEM((NBUF, K), jnp.int32),
          pltpu.VMEM((NBUF, K), jnp.float32),
          pltpu.VMEM((NBUF, K, d), jnp.float32),
          pltpu.VMEM_SHARED((n, d), jnp.float32),
          [pltpu.SemaphoreType.DMA for _ in range(NBUF)],
      ],
      compiler_params=pltpu.CompilerParams(use_tc_tiling_on_sc=False),
  )
  def spmm(h_hbm, cols_hbm, rows_hbm, vals_hbm, out_hbm,
           col_v, dst_v, val_v, gat, acc_sh, sem_g):
    cid = lax.axis_index("c")
    sid = lax.axis_index("s")
    wid = cid * ns + sid

    pltpu.sync_copy(cols_hbm.at[wid], col_v)

    zeros = jnp.zeros((_L,), jnp.float32)

    def zrow(r, _):
      for j in range(d // _L):
        gat[0, r, pl.ds(j * _L, _L)] = zeros
      return 0

    lax.fori_loop(0, ROWCH, zrow, 0)
    for k in range(max_k):
      t = k * ns + sid

      @pl.when(t < n_row_chunks)
      def _():
        pltpu.sync_copy(
            gat.at[0, pl.ds(0, ROWCH)], acc_sh.at[pl.ds(t * ROWCH, ROWCH)])

    pltpu.async_copy(h_hbm.at[col_v.at[0]], gat.at[0], sem_g[0])
    pltpu.async_copy(rows_hbm.at[wid, 0], dst_v.at[0], sem_g[0])
    pltpu.async_copy(vals_hbm.at[wid, 0], val_v.at[0], sem_g[0])
    plsc.subcore_barrier()

    def scale(u):
      for g in range(K // _L):
        a16 = val_v[u, pl.ds(g * _L, _L)]
        for j in range(_L):
          ei = g * _L + j
          a = a16[j]
          for jj in range(d // _L):
            sl = pl.ds(jj * _L, _L)
            gat[u, ei, sl] = gat[u, ei, sl] * a

    def chunk(i, _):
      b = lax.rem(i, 2)

      @pl.when(jnp.logical_and(i + 1 < n_chunks, b == 0))
      def _():
        pltpu.async_copy(h_hbm.at[col_v.at[i + 1]], gat.at[1], sem_g[1])
        pltpu.async_copy(rows_hbm.at[wid, i + 1], dst_v.at[1], sem_g[1])
        pltpu.async_copy(vals_hbm.at[wid, i + 1], val_v.at[1], sem_g[1])

      @pl.when(jnp.logical_and(i + 1 < n_chunks, b == 1))
      def _():
        pltpu.async_copy(h_hbm.at[col_v.at[i + 1]], gat.at[0], sem_g[0])
        pltpu.async_copy(rows_hbm.at[wid, i + 1], dst_v.at[0], sem_g[0])
        pltpu.async_copy(vals_hbm.at[wid, i + 1], val_v.at[0], sem_g[0])

      @pl.when(b == 0)
      def _():
        pltpu.make_async_copy(
            h_hbm.at[col_v.at[i]], gat.at[0], sem_g[0]).wait()
        pltpu.make_async_copy(
            rows_hbm.at[wid, i], dst_v.at[0], sem_g[0]).wait()
        pltpu.make_async_copy(
            vals_hbm.at[wid, i], val_v.at[0], sem_g[0]).wait()
        scale(0)

      @pl.when(b == 1)
      def _():
        pltpu.make_async_copy(
            h_hbm.at[col_v.at[i]], gat.at[1], sem_g[1]).wait()
        pltpu.make_async_copy(
            rows_hbm.at[wid, i], dst_v.at[1], sem_g[1]).wait()
        pltpu.make_async_copy(
            vals_hbm.at[wid, i], val_v.at[1], sem_g[1]).wait()
        scale(1)

      pltpu.sync_copy(gat.at[b], acc_sh.at[dst_v.at[b]], add=True)
      return 0

    lax.fori_loop(0, n_chunks, chunk, 0)
    plsc.subcore_barrier()

    for k in range(max_k):
      t = k * ns + sid

      @pl.when(t < n_row_chunks)
      def _():
        pltpu.sync_copy(
            acc_sh.at[pl.ds(t * ROWCH, ROWCH)], gat.at[0, pl.ds(0, ROWCH)])
        pltpu.sync_copy(
            gat.at[0, pl.ds(0, ROWCH)], out_hbm.at[cid, pl.ds(t * ROWCH, ROWCH)])

  return spmm


_spmm0 = _make_spmm(N, E, H_DIM)
_spmm1 = _make_spmm(N, E, C)

_MB = 2000


def _mm_body(x_ref, w_ref, o_ref):
  o_ref[...] = jnp.dot(x_ref[...], w_ref[...],
                       preferred_element_type=jnp.float32)


def _relu_mm_body(p0_ref, p1_ref, b_ref, w_ref, o_ref):
  h = jnp.maximum(p0_ref[...] + p1_ref[...] + b_ref[...], 0.0)
  o_ref[...] = jnp.dot(h, w_ref[...], preferred_element_type=jnp.float32)


def _softmax_body(p0_ref, p1_ref, b_ref, o_ref):
  z = p0_ref[...] + p1_ref[...] + b_ref[...]
  z = z - jnp.max(z, axis=-1, keepdims=True)
  ez = jnp.exp(z)
  o_ref[...] = ez / jnp.sum(ez, axis=-1, keepdims=True)


def kernel(X, edge_index, A_values, W0, b0, W1, b1):
  def pad3(x):
    x = x.reshape(32, E // 32)
    pad = jnp.zeros((32, 79 * 128 - E // 32), dtype=x.dtype)
    return jnp.concatenate([x, pad], axis=1).reshape(32, 79, 128)

  rows3 = pad3(edge_index[0])
  cols3 = pad3(edge_index[1])
  vals3 = pad3(A_values)

  h0 = pl.pallas_call(
      _mm_body,
      grid=(N // _MB,),
      in_specs=[
          pl.BlockSpec((_MB, D), lambda i: (i, 0)),
          pl.BlockSpec((D, H_DIM), lambda i: (0, 0)),
      ],
      out_specs=pl.BlockSpec((_MB, H_DIM), lambda i: (i, 0)),
      out_shape=jax.ShapeDtypeStruct((N, H_DIM), jnp.float32),
  )(X, W0)

  p = _spmm0(h0, cols3, rows3, vals3)

  h1 = pl.pallas_call(
      _relu_mm_body,
      grid=(N // _MB,),
      in_specs=[
          pl.BlockSpec((_MB, H_DIM), lambda i: (i, 0)),
          pl.BlockSpec((_MB, H_DIM), lambda i: (i, 0)),
          pl.BlockSpec((1, H_DIM), lambda i: (0, 0)),
          pl.BlockSpec((H_DIM, C), lambda i: (0, 0)),
      ],
      out_specs=pl.BlockSpec((_MB, C), lambda i: (i, 0)),
      out_shape=jax.ShapeDtypeStruct((N, C), jnp.float32),
  )(p[0], p[1], b0.reshape(1, H_DIM), W1)

  q = _spmm1(h1, cols3, rows3, vals3)

  out = pl.pallas_call(
      _softmax_body,
      grid=(N // _MB,),
      in_specs=[
          pl.BlockSpec((_MB, C), lambda i: (i, 0)),
          pl.BlockSpec((_MB, C), lambda i: (i, 0)),
          pl.BlockSpec((1, C), lambda i: (0, 0)),
      ],
      out_specs=pl.BlockSpec((_MB, C), lambda i: (i, 0)),
      out_shape=jax.ShapeDtypeStruct((N, C), jnp.float32),
  )(q[0], q[1], b1.reshape(1, C))

  return out


# speedup vs baseline: 1.6823x; 1.6823x over previous
import functools

import jax
import jax.numpy as jnp
from jax import lax
from jax.experimental import pallas as pl
from jax.experimental.pallas import tpu as pltpu
from jax.experimental.pallas import tpu_sc as plsc

N = 10000
E = 320000
D = 128
H_DIM = 128
C = 64

_L = 16


def _make_spmm(n, e, d):
  info = plsc.get_sparse_core_info()
  nc, ns = info.num_cores, info.num_subcores
  nw = nc * ns
  e_per_w = e // nw
  K = 80
  n_chunks = e_per_w // K
  NBUF = 2
  ROWCH = K
  n_row_chunks = n // ROWCH
  max_k = -(-n_row_chunks // ns)

  mesh = plsc.VectorSubcoreMesh(core_axis_name="c", subcore_axis_name="s")

  @functools.partial(
      pl.kernel,
      out_type=jax.ShapeDtypeStruct((nc, n, d), jnp.float32),
      mesh=mesh,
      scratch_types=[
          pltpu.VMEM((n_chunks, K), jnp.int32),
          pltpu.VMEM((n_chunks, K), jnp.int32),
          pltpu.VMEM((NBUF, K), jnp.float32),
          pltpu.VMEM((NBUF, K, d), jnp.float32),
          pltpu.VMEM_SHARED((n, d), jnp.float32),
          [pltpu.SemaphoreType.DMA for _ in range(NBUF)],
      ],
      compiler_params=pltpu.CompilerParams(use_tc_tiling_on_sc=False),
  )
  def spmm(h_hbm, cols_hbm, rows_hbm, vals_hbm, out_hbm,
           col_v, dst_v, val_v, gat, acc_sh, sem_g):
    cid = lax.axis_index("c")
    sid = lax.axis_index("s")
    wid = cid * ns + sid

    pltpu.sync_copy(cols_hbm.at[wid], col_v)
    pltpu.sync_copy(rows_hbm.at[wid], dst_v)

    zeros = jnp.zeros((_L,), jnp.float32)

    def zrow(r, _):
      for j in range(d // _L):
        gat[0, r, pl.ds(j * _L, _L)] = zeros
      return 0

    lax.fori_loop(0, ROWCH, zrow, 0)
    for k in range(max_k):
      t = k * ns + sid

      @pl.when(t < n_row_chunks)
      def _():
        pltpu.sync_copy(gat.at[0], acc_sh.at[pl.ds(t * ROWCH, ROWCH)])

    pltpu.async_copy(h_hbm.at[col_v.at[0]], gat.at[0], sem_g[0])
    pltpu.async_copy(vals_hbm.at[wid, 0], val_v.at[0], sem_g[0])
    plsc.subcore_barrier()

    def scale(u):
      for g in range(K // _L):
        a16 = val_v[u, pl.ds(g * _L, _L)]
        for j in range(_L):
          ei = g * _L + j
          a = a16[j]
          for jj in range(d // _L):
            sl = pl.ds(jj * _L, _L)
            gat[u, ei, sl] = gat[u, ei, sl] * a

    def chunk(i, _):
      b = lax.rem(i, 2)

      @pl.when(jnp.logical_and(i + 1 < n_chunks, b == 0))
      def _():
        pltpu.async_copy(h_hbm.at[col_v.at[i + 1]], gat.at[1], sem_g[1])
        pltpu.async_copy(vals_hbm.at[wid, i + 1], val_v.at[1], sem_g[1])

      @pl.when(jnp.logical_and(i + 1 < n_chunks, b == 1))
      def _():
        pltpu.async_copy(h_hbm.at[col_v.at[i + 1]], gat.at[0], sem_g[0])
        pltpu.async_copy(vals_hbm.at[wid, i + 1], val_v.at[0], sem_g[0])

      @pl.when(b == 0)
      def _():
        pltpu.make_async_copy(
            h_hbm.at[col_v.at[i]], gat.at[0], sem_g[0]).wait()
        pltpu.make_async_copy(
            vals_hbm.at[wid, i], val_v.at[0], sem_g[0]).wait()
        scale(0)

      @pl.when(b == 1)
      def _():
        pltpu.make_async_copy(
            h_hbm.at[col_v.at[i]], gat.at[1], sem_g[1]).wait()
        pltpu.make_async_copy(
            vals_hbm.at[wid, i], val_v.at[1], sem_g[1]).wait()
        scale(1)

      pltpu.sync_copy(gat.at[b], acc_sh.at[dst_v.at[i]], add=True)
      return 0

    lax.fori_loop(0, n_chunks, chunk, 0)
    plsc.subcore_barrier()

    for k in range(max_k):
      t = k * ns + sid

      @pl.when(t < n_row_chunks)
      def _():
        pltpu.sync_copy(acc_sh.at[pl.ds(t * ROWCH, ROWCH)], gat.at[0])
        pltpu.sync_copy(gat.at[0], out_hbm.at[cid, pl.ds(t * ROWCH, ROWCH)])

  return spmm


_spmm0 = _make_spmm(N, E, H_DIM)
_spmm1 = _make_spmm(N, E, C)

_MB = 2000


def _mm_body(x_ref, w_ref, o_ref):
  o_ref[...] = jnp.dot(x_ref[...], w_ref[...],
                       preferred_element_type=jnp.float32)


def _relu_mm_body(p0_ref, p1_ref, b_ref, w_ref, o_ref):
  h = jnp.maximum(p0_ref[...] + p1_ref[...] + b_ref[...], 0.0)
  o_ref[...] = jnp.dot(h, w_ref[...], preferred_element_type=jnp.float32)


def _softmax_body(p0_ref, p1_ref, b_ref, o_ref):
  z = p0_ref[...] + p1_ref[...] + b_ref[...]
  z = z - jnp.max(z, axis=-1, keepdims=True)
  ez = jnp.exp(z)
  o_ref[...] = ez / jnp.sum(ez, axis=-1, keepdims=True)


def kernel(X, edge_index, A_values, W0, b0, W1, b1):
  rows3 = edge_index[0].reshape(32, 125, 80)
  cols3 = edge_index[1].reshape(32, 125, 80)
  vals3 = A_values.reshape(32, 125, 80)

  h0 = pl.pallas_call(
      _mm_body,
      grid=(N // _MB,),
      in_specs=[
          pl.BlockSpec((_MB, D), lambda i: (i, 0)),
          pl.BlockSpec((D, H_DIM), lambda i: (0, 0)),
      ],
      out_specs=pl.BlockSpec((_MB, H_DIM), lambda i: (i, 0)),
      out_shape=jax.ShapeDtypeStruct((N, H_DIM), jnp.float32),
  )(X, W0)

  p = _spmm0(h0, cols3, rows3, vals3)

  h1 = pl.pallas_call(
      _relu_mm_body,
      grid=(N // _MB,),
      in_specs=[
          pl.BlockSpec((_MB, H_DIM), lambda i: (i, 0)),
          pl.BlockSpec((_MB, H_DIM), lambda i: (i, 0)),
          pl.BlockSpec((1, H_DIM), lambda i: (0, 0)),
          pl.BlockSpec((H_DIM, C), lambda i: (0, 0)),
      ],
      out_specs=pl.BlockSpec((_MB, C), lambda i: (i, 0)),
      out_shape=jax.ShapeDtypeStruct((N, C), jnp.float32),
  )(p[0], p[1], b0.reshape(1, H_DIM), W1)

  q = _spmm1(h1, cols3, rows3, vals3)

  out = pl.pallas_call(
      _softmax_body,
      grid=(N // _MB,),
      in_specs=[
          pl.BlockSpec((_MB, C), lambda i: (i, 0)),
          pl.BlockSpec((_MB, C), lambda i: (i, 0)),
          pl.BlockSpec((1, C), lambda i: (0, 0)),
      ],
      out_specs=pl.BlockSpec((_MB, C), lambda i: (i, 0)),
      out_shape=jax.ShapeDtypeStruct((N, C), jnp.float32),
  )(q[0], q[1], b1.reshape(1, C))

  return out
